# Initial kernel scaffold; baseline (speedup 1.0000x reference)
import jax
import jax.numpy as jnp
from jax.experimental import pallas as pl


def kernel(x, edge_index, edge_attr, batch, group_emb, hemi_emb, W1_0, b1_0, gamma_0, beta_0, W2_0, b2_0, W1_1, b1_1, gamma_1, beta_1, W2_1, b2_1, W1_2, b1_2, gamma_2, beta_2, W2_2, b2_2, Wm1, bm1, Wm2, bm2):
    raise NotImplementedError("write your pallas kernel here")


# speedup vs baseline: 4.0649x; 4.0649x over previous
import functools

import jax
import jax.numpy as jnp
from jax import lax
from jax.experimental import pallas as pl
from jax.experimental.pallas import tpu as pltpu
from jax.experimental.pallas import tpu_sc as plsc

N = 10000
E = 320000
D = 128
H = 128
OUT = 8
NGRAPHS = 64

NC = 2
NS = 16
NW = NC * NS
LANES = 16

NP = 10240
CH = 128
EPW = 10112
EPAD = NW * EPW
NCHUNK = EPW // CH
RPT = NP // NS
ZR = 64


@functools.lru_cache(maxsize=None)
def _make_seg_sum(Dp):
    mesh = plsc.VectorSubcoreMesh(core_axis_name="c", subcore_axis_name="s",
                                  num_cores=NC, num_subcores=NS)

    @functools.partial(
        pl.kernel,
        out_type=jax.ShapeDtypeStruct((NC, NP, Dp), jnp.float32),
        mesh=mesh,
        scratch_types=[
            pltpu.VMEM((CH,), jnp.int32),
            pltpu.VMEM((CH,), jnp.int32),
            pltpu.VMEM((CH, Dp), jnp.float32),
            pltpu.VMEM((ZR, Dp), jnp.float32),
            pltpu.VMEM_SHARED((NP, Dp), jnp.float32),
            pltpu.SemaphoreType.DMA,
        ],
    )
    def seg(h_hbm, src_hbm, dst_hbm, out_hbm, src_v, dst_v, rows_v, zbuf,
            acc, sem):
        c = lax.axis_index("c")
        s = lax.axis_index("s")
        wid = s * NC + c

        def zrow(i, _):
            for j in range(Dp // LANES):
                zbuf[i, pl.ds(j * LANES, LANES)] = jnp.zeros(
                    (LANES,), jnp.float32)
            return _
        lax.fori_loop(0, ZR, zrow, None)

        def zcopy(i, _):
            pltpu.sync_copy(zbuf, acc.at[pl.ds(s * RPT + i * ZR, ZR)])
            return _
        lax.fori_loop(0, RPT // ZR, zcopy, None)
        plsc.subcore_barrier()

        def body(g, _):
            base = wid * EPW + g * CH
            pltpu.sync_copy(src_hbm.at[pl.ds(base, CH)], src_v)
            pltpu.sync_copy(dst_hbm.at[pl.ds(base, CH)], dst_v)
            pltpu.async_copy(h_hbm.at[src_v], rows_v, sem).wait()
            pltpu.sync_copy(rows_v, acc.at[dst_v], add=True)
            return _
        lax.fori_loop(0, NCHUNK, body, None)
        plsc.subcore_barrier()

        pltpu.sync_copy(acc.at[pl.ds(s * RPT, RPT)],
                        out_hbm.at[c, pl.ds(s * RPT, RPT)])

    return seg


def _seg_sum(h, src_p, dst_p, Dp):
    return _make_seg_sum(Dp)(h, src_p, dst_p)


def _embed_body(x_ref, ge_ref, he_ref, W1a_ref, W1b_ref, o_ref):
    n = lax.broadcasted_iota(jnp.int32, (NP, 1), 0)
    gid = jnp.where(n < 16, n // 2, 0)
    onehot_g = (gid == lax.broadcasted_iota(jnp.int32, (NP, 8), 1)
                ).astype(jnp.float32)
    hemi = n % 2
    onehot_h = (hemi == lax.broadcasted_iota(jnp.int32, (NP, 2), 1)
                ).astype(jnp.float32)
    emb_w = jnp.concatenate([
        jnp.dot(ge_ref[...], W1b_ref[0:2, :],
                preferred_element_type=jnp.float32,
                precision=lax.Precision.HIGHEST),
        jnp.dot(he_ref[...], W1b_ref[2:4, :],
                preferred_element_type=jnp.float32,
                precision=lax.Precision.HIGHEST),
    ], axis=0)
    onehot = jnp.concatenate([onehot_g, onehot_h], axis=1)
    u = (jnp.dot(x_ref[...], W1a_ref[...],
                 preferred_element_type=jnp.float32,
                 precision=lax.Precision.HIGHEST) +
         jnp.dot(onehot, emb_w, preferred_element_type=jnp.float32,
                 precision=lax.Precision.HIGHEST))
    mask = (n < N).astype(jnp.float32)
    o_ref[...] = u * mask


def _embed(x_p, group_emb, hemi_emb, W1a, W1b):
    return pl.pallas_call(
        _embed_body,
        out_shape=jax.ShapeDtypeStruct((NP, H), jnp.float32),
    )(x_p, group_emb, hemi_emb, W1a, W1b)


def _bn_relu_mm(y, gamma, beta, W2, b2, mask):
    y = y * mask
    mu = jnp.sum(y, axis=0, keepdims=True) / N
    var = jnp.sum(y * y, axis=0, keepdims=True) / N - mu * mu
    y = gamma * (y - mu) / jnp.sqrt(var + 1e-5) + beta
    y = jnp.maximum(y, 0.0) * mask
    o = jnp.dot(y, W2, preferred_element_type=jnp.float32,
                precision=lax.Precision.HIGHEST) + b2
    return jnp.maximum(o, 0.0) * mask


def _layer_math(h, agg0, agg1, W1, b1, gamma, beta, W2, b2):
    mask = (lax.broadcasted_iota(jnp.int32, (NP, 1), 0) < N).astype(
        jnp.float32)
    z = h + agg0 + agg1
    y = jnp.dot(z, W1, preferred_element_type=jnp.float32,
                precision=lax.Precision.HIGHEST) + b1
    return _bn_relu_mm(y, gamma, beta, W2, b2, mask)


def _dense0_body(u_ref, a_ref, b1_ref, g_ref, be_ref, W2_ref, b2_ref, o_ref):
    mask = (lax.broadcasted_iota(jnp.int32, (NP, 1), 0) < N).astype(
        jnp.float32)
    y = u_ref[...] + a_ref[0] + a_ref[1] + b1_ref[...]
    o_ref[...] = _bn_relu_mm(y, g_ref[...], be_ref[...], W2_ref[...],
                             b2_ref[...], mask)


def _dense0(u, agg, b1, gamma, beta, W2, b2):
    return pl.pallas_call(
        _dense0_body,
        out_shape=jax.ShapeDtypeStruct((NP, H), jnp.float32),
    )(u, agg, b1.reshape(1, H), gamma.reshape(1, H), beta.reshape(1, H),
      W2, b2.reshape(1, H))


def _dense_body(h_ref, a_ref, W1_ref, b1_ref, g_ref, be_ref, W2_ref, b2_ref,
                o_ref):
    o_ref[...] = _layer_math(h_ref[...], a_ref[0], a_ref[1], W1_ref[...],
                             b1_ref[...], g_ref[...], be_ref[...],
                             W2_ref[...], b2_ref[...])


def _dense(h, agg, W1, b1, gamma, beta, W2, b2):
    return pl.pallas_call(
        _dense_body,
        out_shape=jax.ShapeDtypeStruct((NP, H), jnp.float32),
    )(h, agg, W1, b1.reshape(1, H), gamma.reshape(1, H), beta.reshape(1, H),
      W2, b2.reshape(1, H))


def _final_body(h_ref, a_ref, W1_ref, b1_ref, g_ref, be_ref, W2_ref, b2_ref,
                batch_ref, Wm1_ref, bm1_ref, Wm2_ref, bm2_ref, o_ref):
    h3 = _layer_math(h_ref[...], a_ref[0], a_ref[1], W1_ref[...],
                     b1_ref[...], g_ref[...], be_ref[...], W2_ref[...],
                     b2_ref[...])
    onehot = (batch_ref[...] ==
              lax.broadcasted_iota(jnp.int32, (NP, NGRAPHS), 1)
              ).astype(jnp.float32)
    pooled = lax.dot_general(onehot, h3, (((0,), (0,)), ((), ())),
                             preferred_element_type=jnp.float32,
                             precision=lax.Precision.HIGHEST)
    y = jnp.maximum(
        jnp.dot(pooled, Wm1_ref[...], preferred_element_type=jnp.float32,
                precision=lax.Precision.HIGHEST) + bm1_ref[...], 0.0)
    o_ref[...] = jnp.dot(y, Wm2_ref[...], preferred_element_type=jnp.float32,
                         precision=lax.Precision.HIGHEST) + bm2_ref[...]


def _final(h, agg, W1, b1, gamma, beta, W2, b2, batch_p, Wm1, bm1, Wm2, bm2):
    return pl.pallas_call(
        _final_body,
        out_shape=jax.ShapeDtypeStruct((NGRAPHS, OUT), jnp.float32),
    )(h, agg, W1, b1.reshape(1, H), gamma.reshape(1, H), beta.reshape(1, H),
      W2, b2.reshape(1, H), batch_p, Wm1, bm1.reshape(1, H), Wm2,
      bm2.reshape(1, OUT))


def kernel(x, edge_index, edge_attr, batch, group_emb, hemi_emb,
           W1_0, b1_0, gamma_0, beta_0, W2_0, b2_0,
           W1_1, b1_1, gamma_1, beta_1, W2_1, b2_1,
           W1_2, b1_2, gamma_2, beta_2, W2_2, b2_2,
           Wm1, bm1, Wm2, bm2):
    src = edge_index[0]
    dst = edge_index[1]
    pad_e = EPAD - E
    src_p = jnp.concatenate([src, jnp.zeros((pad_e,), jnp.int32)])
    dst_p = jnp.concatenate([dst, jnp.full((pad_e,), NP - 1, jnp.int32)])
    x_p = jnp.pad(x, ((0, NP - N), (0, 0)))
    batch_p = jnp.pad(batch, (0, NP - N),
                      constant_values=NGRAPHS).reshape(NP, 1)

    u0 = _embed(x_p, group_emb, hemi_emb, W1_0[:D], W1_0[D:])
    agg0 = _seg_sum(u0, src_p, dst_p, H)
    h1 = _dense0(u0, agg0, b1_0, gamma_0, beta_0, W2_0, b2_0)
    agg1 = _seg_sum(h1, src_p, dst_p, H)
    h2 = _dense(h1, agg1, W1_1, b1_1, gamma_1, beta_1, W2_1, b2_1)
    agg2 = _seg_sum(h2, src_p, dst_p, H)
    return _final(h2, agg2, W1_2, b1_2, gamma_2, beta_2, W2_2, b2_2,
                  batch_p, Wm1, bm1, Wm2, bm2)
